# baseline (speedup 1.0000x reference)
import functools

import jax
import jax.numpy as jnp
from jax import lax
from jax.experimental import pallas as pl
from jax.experimental.pallas import tpu as pltpu
from jax.experimental.pallas import tpu_sc as plsc

BATCH = 16384
SLOTS = 26
FID_DIMS = 4
PACK = 8
GROUP = 16
GROUP_B_LO = SLOTS - GROUP
LANES = 128
NROWS = BATCH * GROUP // LANES
NWORKERS = 32
ROWS_PER_W = NROWS // NWORKERS
SAMP_PER_W = BATCH // NWORKERS
DEPTH = 4


def _sc_gather_body(fids_hbm, t8_hbm, outa_hbm, outb_hbm,
                    fids_v, idxa_v, idxb_v, dst_v, sem):
    wid = lax.axis_index("s") * 2 + lax.axis_index("c")
    base = wid * ROWS_PER_W
    pltpu.sync_copy(fids_hbm.at[pl.ds(wid * SAMP_PER_W, SAMP_PER_W)], fids_v)

    def build(j, carry):
        for q in range(8):
            s = 8 * j + q
            idxa_v[j, pl.ds(16 * q, 16)] = fids_v[s, pl.ds(0, GROUP)]
            idxb_v[j, pl.ds(16 * q, 16)] = fids_v[s, pl.ds(GROUP_B_LO, GROUP)]
        return carry

    lax.fori_loop(0, ROWS_PER_W, build, 0)

    for idx_v, out_hbm in ((idxa_v, outa_hbm), (idxb_v, outb_hbm)):

        def wait_for(j, idx_v=idx_v):
            pltpu.make_async_copy(t8_hbm.at[idx_v.at[j]], dst_v.at[j],
                                  sem).wait()

        def fire(j, carry, idx_v=idx_v, wait_for=wait_for):
            pltpu.async_copy(t8_hbm.at[idx_v.at[j]], dst_v.at[j], sem)

            @pl.when(j >= DEPTH)
            def _():
                wait_for(j - DEPTH)

            return carry

        lax.fori_loop(0, ROWS_PER_W, fire, 0)

        def drain(j, carry, wait_for=wait_for):
            wait_for(j)
            return carry

        lax.fori_loop(ROWS_PER_W - DEPTH, ROWS_PER_W, drain, 0)
        pltpu.sync_copy(dst_v, out_hbm.at[pl.ds(base, ROWS_PER_W)])


@functools.cache
def _sc_gather():
    return functools.partial(
        pl.kernel,
        out_type=(
            jax.ShapeDtypeStruct((NROWS, LANES, PACK), jnp.float32),
            jax.ShapeDtypeStruct((NROWS, LANES, PACK), jnp.float32),
        ),
        mesh=plsc.VectorSubcoreMesh(core_axis_name="c", subcore_axis_name="s",
                                    num_cores=2, num_subcores=16),
        scratch_types=[
            pltpu.VMEM((SAMP_PER_W, SLOTS), jnp.int32),
            pltpu.VMEM((ROWS_PER_W, LANES), jnp.int32),
            pltpu.VMEM((ROWS_PER_W, LANES), jnp.int32),
            pltpu.VMEM((ROWS_PER_W, LANES, PACK), jnp.float32),
            pltpu.SemaphoreType.DMA,
        ],
        compiler_params=pltpu.CompilerParams(use_tc_tiling_on_sc=False),
    )(_sc_gather_body)


BLK = 2048
GW = GROUP * PACK


def _mlp_body(xa_ref, xb_ref, w1a_ref, w1b_ref, b1_ref, w2t_ref, b2_ref,
              w3t_ref, b3_ref, out_ref):
    p = jnp.dot(xa_ref[...], w1a_ref[...], preferred_element_type=jnp.float32)
    p = p + jnp.dot(xb_ref[...], w1b_ref[...],
                    preferred_element_type=jnp.float32)
    h = jnp.maximum(p[:, :16] + b1_ref[...], 0.0)
    bias_mean = p[:, 16]
    h = jnp.dot(h, w2t_ref[...], preferred_element_type=jnp.float32)
    h = jnp.maximum(h + b2_ref[...], 0.0)
    nn = jnp.dot(h, w3t_ref[...], preferred_element_type=jnp.float32)
    out_ref[...] = bias_mean + nn[:, 0] + b3_ref[0, 0]


def _mlp_call(xa, xb, w1a, w1b, b1, w2t, b2, w3t, b3):
    grid = BATCH // BLK
    return pl.pallas_call(
        _mlp_body,
        grid=(grid,),
        in_specs=[
            pl.BlockSpec((BLK, GW), lambda i: (i, 0)),
            pl.BlockSpec((BLK, GW), lambda i: (i, 0)),
            pl.BlockSpec((GW, 17), lambda i: (0, 0)),
            pl.BlockSpec((GW, 17), lambda i: (0, 0)),
            pl.BlockSpec((1, 16), lambda i: (0, 0)),
            pl.BlockSpec((16, 8), lambda i: (0, 0)),
            pl.BlockSpec((1, 8), lambda i: (0, 0)),
            pl.BlockSpec((8, 1), lambda i: (0, 0)),
            pl.BlockSpec((1, 1), lambda i: (0, 0)),
        ],
        out_specs=pl.BlockSpec((BLK,), lambda i: (i,)),
        out_shape=jax.ShapeDtypeStruct((BATCH,), jnp.float32),
    )(xa, xb, w1a, w1b, b1, w2t, b2, w3t, b3)


def _expand_w1(W1):
    w = W1.T.reshape(SLOTS, FID_DIMS, 16)
    w = jnp.concatenate(
        [w, jnp.zeros((SLOTS, PACK - FID_DIMS, 16), jnp.float32)], axis=1)
    e = jnp.zeros((SLOTS, PACK, 1), jnp.float32).at[:, FID_DIMS, 0].set(
        1.0 / SLOTS)
    we = jnp.concatenate([w, e], axis=2)
    wa = we[:GROUP]
    wb = we[GROUP_B_LO:].at[:GROUP - GROUP_B_LO].set(0.0)
    return wa.reshape(GW, 17), wb.reshape(GW, 17)


def kernel(fids_batch, table_w, table_b, W1, b1, W2, b2, W3, b3):
    fids = fids_batch.astype(jnp.int32)
    t8 = jnp.pad(table_w, ((0, 0), (0, PACK - FID_DIMS))).at[:, FID_DIMS].set(
        table_b)
    rows_a, rows_b = _sc_gather()(fids, t8)
    xa = rows_a.reshape(BATCH, GW)
    xb = rows_b.reshape(BATCH, GW)
    w1a, w1b = _expand_w1(W1)
    return _mlp_call(
        xa, xb, w1a, w1b,
        b1.reshape(1, 16),
        W2.T, b2.reshape(1, 8),
        W3.T, b3.reshape(1, 1),
    )


# speedup vs baseline: 5.7984x; 5.7984x over previous
import functools

import jax
import jax.numpy as jnp
from jax import lax
from jax.experimental import pallas as pl
from jax.experimental.pallas import tpu as pltpu
from jax.experimental.pallas import tpu_sc as plsc

BATCH = 16384
SLOTS = 26
FID_DIMS = 4
PACK = 8
GROUP = 16
GROUP_B_LO = SLOTS - GROUP
LANES = 128
NROWS = BATCH * GROUP // LANES
NWORKERS = 32
ROWS_PER_W = NROWS // NWORKERS
SAMP_PER_W = BATCH // NWORKERS
DEPTH = 4


def _sc_gather_body(fids_hbm, t8_hbm, outa_hbm, outb_hbm,
                    fids_v, idxa_v, idxb_v, dst_v, sem):
    wid = lax.axis_index("s") * 2 + lax.axis_index("c")
    base = wid * ROWS_PER_W
    pltpu.sync_copy(fids_hbm.at[pl.ds(wid * SAMP_PER_W, SAMP_PER_W)], fids_v)

    def build(j, carry):
        for q in range(8):
            s = 8 * j + q
            idxa_v[j, pl.ds(16 * q, 16)] = fids_v[s, pl.ds(0, GROUP)]
            idxb_v[j, pl.ds(16 * q, 16)] = fids_v[s, pl.ds(GROUP_B_LO, GROUP)]
        return carry

    lax.fori_loop(0, ROWS_PER_W, build, 0)

    for idx_v, out_hbm in ((idxa_v, outa_hbm), (idxb_v, outb_hbm)):

        def wait_for(j, idx_v=idx_v):
            pltpu.make_async_copy(t8_hbm.at[idx_v.at[j]], dst_v.at[j],
                                  sem).wait()

        def fire(j, carry, idx_v=idx_v, wait_for=wait_for):
            pltpu.async_copy(t8_hbm.at[idx_v.at[j]], dst_v.at[j], sem)

            @pl.when(j >= DEPTH)
            def _():
                wait_for(j - DEPTH)

            return carry

        lax.fori_loop(0, ROWS_PER_W, fire, 0)

        def drain(j, carry, wait_for=wait_for):
            wait_for(j)
            return carry

        lax.fori_loop(ROWS_PER_W - DEPTH, ROWS_PER_W, drain, 0)
        pltpu.sync_copy(dst_v, out_hbm.at[pl.ds(base, ROWS_PER_W)])


@functools.cache
def _sc_gather():
    return functools.partial(
        pl.kernel,
        out_type=(
            jax.ShapeDtypeStruct((NROWS, LANES, PACK), jnp.float32),
            jax.ShapeDtypeStruct((NROWS, LANES, PACK), jnp.float32),
        ),
        mesh=plsc.VectorSubcoreMesh(core_axis_name="c", subcore_axis_name="s",
                                    num_cores=2, num_subcores=16),
        scratch_types=[
            pltpu.VMEM((SAMP_PER_W, SLOTS), jnp.int32),
            pltpu.VMEM((ROWS_PER_W, LANES), jnp.int32),
            pltpu.VMEM((ROWS_PER_W, LANES), jnp.int32),
            pltpu.VMEM((ROWS_PER_W, LANES, PACK), jnp.float32),
            pltpu.SemaphoreType.DMA,
        ],
        compiler_params=pltpu.CompilerParams(use_tc_tiling_on_sc=False),
    )(_sc_gather_body)


BLK = 2048
GW = GROUP * PACK


def _mlp_body(xa_ref, xb_ref, w1a_ref, w1b_ref, b1_ref, w2t_ref, b2_ref,
              w3t_ref, b3_ref, out_ref):
    p = jnp.dot(xa_ref[...], w1a_ref[...], preferred_element_type=jnp.float32)
    p = p + jnp.dot(xb_ref[...], w1b_ref[...],
                    preferred_element_type=jnp.float32)
    h = jnp.maximum(p[:, :16] + b1_ref[...], 0.0)
    bias_mean = p[:, 16]
    h = jnp.dot(h, w2t_ref[...], preferred_element_type=jnp.float32)
    h = jnp.maximum(h + b2_ref[...], 0.0)
    nn = jnp.dot(h, w3t_ref[...], preferred_element_type=jnp.float32)
    out_ref[...] = bias_mean + nn[:, 0] + b3_ref[0, 0]


def _mlp_call(xa, xb, w1a, w1b, b1, w2t, b2, w3t, b3):
    grid = BATCH // BLK
    return pl.pallas_call(
        _mlp_body,
        grid=(grid,),
        in_specs=[
            pl.BlockSpec((BLK, GW), lambda i: (i, 0)),
            pl.BlockSpec((BLK, GW), lambda i: (i, 0)),
            pl.BlockSpec((GW, 17), lambda i: (0, 0)),
            pl.BlockSpec((GW, 17), lambda i: (0, 0)),
            pl.BlockSpec((1, 16), lambda i: (0, 0)),
            pl.BlockSpec((16, 8), lambda i: (0, 0)),
            pl.BlockSpec((1, 8), lambda i: (0, 0)),
            pl.BlockSpec((8, 1), lambda i: (0, 0)),
            pl.BlockSpec((1, 1), lambda i: (0, 0)),
        ],
        out_specs=pl.BlockSpec((BLK,), lambda i: (i,)),
        out_shape=jax.ShapeDtypeStruct((BATCH,), jnp.float32),
    )(xa, xb, w1a, w1b, b1, w2t, b2, w3t, b3)


def _expand_w1(W1):
    w = W1.T.reshape(SLOTS, FID_DIMS, 16)
    w = jnp.concatenate(
        [w, jnp.zeros((SLOTS, PACK - FID_DIMS, 16), jnp.float32)], axis=1)
    e = jnp.zeros((SLOTS, PACK, 1), jnp.float32).at[:, FID_DIMS, 0].set(
        1.0 / SLOTS)
    we = jnp.concatenate([w, e], axis=2)
    wa = we[:GROUP]
    wb = we[GROUP_B_LO:].at[:GROUP - GROUP_B_LO].set(0.0)
    return wa.reshape(GW, 17), wb.reshape(GW, 17)


def kernel(fids_batch, table_w, table_b, W1, b1, W2, b2, W3, b3):
    fids = fids_batch.astype(jnp.int32)
    t8 = jnp.concatenate(
        [table_w,
         jnp.broadcast_to(table_b[:, None],
                          (table_w.shape[0], PACK - FID_DIMS))],
        axis=1)
    rows_a, rows_b = _sc_gather()(fids, t8)
    xa = rows_a.reshape(BATCH, GW)
    xb = rows_b.reshape(BATCH, GW)
    w1a, w1b = _expand_w1(W1)
    return _mlp_call(
        xa, xb, w1a, w1b,
        b1.reshape(1, 16),
        W2.T, b2.reshape(1, 8),
        W3.T, b3.reshape(1, 1),
    )
